# baseline (speedup 1.0000x reference)
import functools

import jax
import jax.numpy as jnp
from jax import lax
from jax.experimental import pallas as pl
from jax.experimental.pallas import tpu as pltpu
from jax.experimental.pallas import tpu_sc as plsc

EMB = 64
NC, NS, LANES = 2, 16, 16
NW = NC * NS
CHUNK = 160
GROUPS = CHUNK // LANES
NBUF = 2
RUNROLL = 4


@functools.partial(jax.jit, static_argnames=("tot",))
def _run_sc(u_table, i_table, uk, ik, *, tot):
    npw = tot // NW
    nchunk = npw // CHUNK
    mesh = plsc.VectorSubcoreMesh(core_axis_name="c", subcore_axis_name="s")

    idx_t = pltpu.VMEM((nchunk, CHUNK), jnp.int32)
    rows_t = pltpu.VMEM((CHUNK, EMB), jnp.float32)

    @functools.partial(
        pl.kernel,
        out_type=jax.ShapeDtypeStruct((tot,), jnp.float32),
        mesh=mesh,
        compiler_params=pltpu.CompilerParams(
            needs_layout_passes=False, use_tc_tiling_on_sc=True),
        scratch_types=(
            [idx_t] * 2
            + [pltpu.VMEM((npw,), jnp.float32)]
            + [rows_t] * (2 * NBUF)
            + [pltpu.SemaphoreType.DMA] * (2 * NBUF)
        ),
    )
    def sc_kernel(u_tab, i_tab, uk_hbm, ik_hbm, out_hbm,
                  uk_v, ik_v, out_v, *bufsems):
        bufs_u = bufsems[0:NBUF]
        bufs_i = bufsems[NBUF:2 * NBUF]
        sems_u = bufsems[2 * NBUF:3 * NBUF]
        sems_i = bufsems[3 * NBUF:4 * NBUF]
        wid = lax.axis_index("s") * NC + lax.axis_index("c")
        pltpu.sync_copy(uk_hbm.at[wid], uk_v)
        pltpu.sync_copy(ik_hbm.at[wid], ik_v)

        def start(k, b):
            def issue(t, c1):
                uvec = uk_v[k, pl.ds(t * LANES, LANES)]
                ivec = ik_v[k, pl.ds(t * LANES, LANES)]
                for rr in range(LANES):
                    r = t * LANES + rr
                    pltpu.async_copy(
                        u_tab.at[pl.ds(uvec[rr], 1)],
                        bufs_u[b].at[pl.ds(r, 1)], sems_u[b])
                    pltpu.async_copy(
                        i_tab.at[pl.ds(ivec[rr], 1)],
                        bufs_i[b].at[pl.ds(r, 1)], sems_i[b])
                return c1
            lax.fori_loop(0, CHUNK // LANES, issue, 0)

        def drain(b):
            pltpu.make_async_copy(
                u_tab.at[pl.ds(0, CHUNK)], bufs_u[b], sems_u[b]).wait()
            pltpu.make_async_copy(
                i_tab.at[pl.ds(0, CHUNK)], bufs_i[b], sems_i[b]).wait()

        for b in range(NBUF):
            start(b, b)

        lane = lax.iota(jnp.int32, LANES)

        def pair_body(p, carry):
            for b in range(NBUF):
                k = p * NBUF + b
                drain(b)

                def group_body(g, c2):
                    jvec = lane + g * LANES
                    zero = jnp.zeros((LANES,), jnp.float32)

                    def dblock(t, accs):
                        prods = []
                        for dd in range(8):
                            skew = (lane + t * 8 + dd) & (EMB - 1)
                            uv = plsc.load_gather(bufs_u[b], [jvec, skew])
                            iv = plsc.load_gather(bufs_i[b], [jvec, skew])
                            prods.append(uv * iv)
                        return tuple(a + p2[0] + p2[1] for a, p2 in
                                     zip(accs, zip(prods[0::2], prods[1::2])))

                    accs = lax.fori_loop(
                        0, EMB // 8, dblock, (zero, zero, zero, zero))
                    acc = (accs[0] + accs[1]) + (accs[2] + accs[3])
                    out_v[pl.ds(k * CHUNK + g * LANES, LANES)] = acc
                    return c2

                lax.fori_loop(0, GROUPS, group_body, 0)

                nk = k + NBUF

                @pl.when(nk < nchunk)
                def _():
                    start(nk, b)
            return carry

        lax.fori_loop(0, nchunk // NBUF, pair_body, 0)
        pltpu.sync_copy(out_v, out_hbm.at[pl.ds(wid * npw, npw)])

    return sc_kernel(u_table, i_table, uk, ik)


def kernel(data, u_table, i_table):
    b, s, _ = data.shape
    tot = b * s
    nchunk = tot // NW // CHUNK
    flat = data.reshape(tot, 2).astype(jnp.int32)
    uk = flat[:, 0].reshape(NW, nchunk, CHUNK)
    ik = flat[:, 1].reshape(NW, nchunk, CHUNK)
    out = _run_sc(u_table, i_table, uk, ik, tot=tot)
    return out.reshape(b, s)


# speedup vs baseline: 1.0150x; 1.0150x over previous
import functools

import jax
import jax.numpy as jnp
from jax import lax
from jax.experimental import pallas as pl
from jax.experimental.pallas import tpu as pltpu
from jax.experimental.pallas import tpu_sc as plsc

EMB = 64
NC, NS, LANES = 2, 16, 16
NW = NC * NS
CHUNK = 80
GROUPS = CHUNK // LANES
NBUF = 4
RUNROLL = 4


@functools.partial(jax.jit, static_argnames=("tot",))
def _run_sc(u_table, i_table, uk, ik, *, tot):
    npw = tot // NW
    nchunk = npw // CHUNK
    mesh = plsc.VectorSubcoreMesh(core_axis_name="c", subcore_axis_name="s")

    idx_t = pltpu.VMEM((nchunk, CHUNK), jnp.int32)
    rows_t = pltpu.VMEM((CHUNK, EMB), jnp.float32)

    @functools.partial(
        pl.kernel,
        out_type=jax.ShapeDtypeStruct((tot,), jnp.float32),
        mesh=mesh,
        compiler_params=pltpu.CompilerParams(
            needs_layout_passes=False, use_tc_tiling_on_sc=True),
        scratch_types=(
            [idx_t] * 2
            + [pltpu.VMEM((npw,), jnp.float32)]
            + [rows_t] * (2 * NBUF)
            + [pltpu.SemaphoreType.DMA] * (2 * NBUF)
        ),
    )
    def sc_kernel(u_tab, i_tab, uk_hbm, ik_hbm, out_hbm,
                  uk_v, ik_v, out_v, *bufsems):
        bufs_u = bufsems[0:NBUF]
        bufs_i = bufsems[NBUF:2 * NBUF]
        sems_u = bufsems[2 * NBUF:3 * NBUF]
        sems_i = bufsems[3 * NBUF:4 * NBUF]
        wid = lax.axis_index("s") * NC + lax.axis_index("c")
        pltpu.sync_copy(uk_hbm.at[wid], uk_v)
        pltpu.sync_copy(ik_hbm.at[wid], ik_v)

        def start(k, b):
            def issue(t, c1):
                uvec = uk_v[k, pl.ds(t * LANES, LANES)]
                ivec = ik_v[k, pl.ds(t * LANES, LANES)]
                for rr in range(LANES):
                    r = t * LANES + rr
                    pltpu.async_copy(
                        u_tab.at[pl.ds(uvec[rr], 1)],
                        bufs_u[b].at[pl.ds(r, 1)], sems_u[b])
                    pltpu.async_copy(
                        i_tab.at[pl.ds(ivec[rr], 1)],
                        bufs_i[b].at[pl.ds(r, 1)], sems_i[b])
                return c1
            lax.fori_loop(0, CHUNK // LANES, issue, 0)

        def drain(b):
            pltpu.make_async_copy(
                u_tab.at[pl.ds(0, CHUNK)], bufs_u[b], sems_u[b]).wait()
            pltpu.make_async_copy(
                i_tab.at[pl.ds(0, CHUNK)], bufs_i[b], sems_i[b]).wait()

        for b in range(2):
            start(b, b)

        lane = lax.iota(jnp.int32, LANES)

        def pair_body(p, carry):
            for b in range(NBUF):
                k = p * NBUF + b
                tb = (b + 2) % NBUF
                drain(b)
                nk = k + 2

                def group_body(g, c2):
                    @pl.when(nk < nchunk)
                    def _():
                        uvec = uk_v[nk, pl.ds(g * LANES, LANES)]
                        ivec = ik_v[nk, pl.ds(g * LANES, LANES)]
                        for rr in range(LANES):
                            r = g * LANES + rr
                            pltpu.async_copy(
                                u_tab.at[pl.ds(uvec[rr], 1)],
                                bufs_u[tb].at[pl.ds(r, 1)], sems_u[tb])
                            pltpu.async_copy(
                                i_tab.at[pl.ds(ivec[rr], 1)],
                                bufs_i[tb].at[pl.ds(r, 1)], sems_i[tb])

                    jvec = lane + g * LANES
                    zero = jnp.zeros((LANES,), jnp.float32)

                    def dblock(t, accs):
                        prods = []
                        for dd in range(8):
                            skew = (lane + t * 8 + dd) & (EMB - 1)
                            uv = plsc.load_gather(bufs_u[b], [jvec, skew])
                            iv = plsc.load_gather(bufs_i[b], [jvec, skew])
                            prods.append(uv * iv)
                        return tuple(a + p2[0] + p2[1] for a, p2 in
                                     zip(accs, zip(prods[0::2], prods[1::2])))

                    accs = lax.fori_loop(
                        0, EMB // 8, dblock, (zero, zero, zero, zero))
                    acc = (accs[0] + accs[1]) + (accs[2] + accs[3])
                    out_v[pl.ds(k * CHUNK + g * LANES, LANES)] = acc
                    return c2

                lax.fori_loop(0, GROUPS, group_body, 0)
            return carry

        lax.fori_loop(0, nchunk // NBUF, pair_body, 0)
        pltpu.sync_copy(out_v, out_hbm.at[pl.ds(wid * npw, npw)])

    return sc_kernel(u_table, i_table, uk, ik)


def kernel(data, u_table, i_table):
    b, s, _ = data.shape
    tot = b * s
    nchunk = tot // NW // CHUNK
    flat = data.reshape(tot, 2).astype(jnp.int32)
    uk = flat[:, 0].reshape(NW, nchunk, CHUNK)
    ik = flat[:, 1].reshape(NW, nchunk, CHUNK)
    out = _run_sc(u_table, i_table, uk, ik, tot=tot)
    return out.reshape(b, s)
